# baseline (speedup 1.0000x reference)
import functools

import numpy as np
import jax
import jax.numpy as jnp
from jax.experimental import pallas as pl
from jax.experimental.pallas import tpu as pltpu
from jax.experimental.pallas import tpu_sc as plsc

_PLACEHOLDER = 265
_B, _N, _D = 128, 77, 768
_T = 4 * _D
_INNER = 512
_KB = 512
_NK = _T // _KB
_HALF = _D // 2


def _emb_kernel(ts_ref, w1_ref, b1_ref, w2_ref, out_ref):
    k = pl.program_id(0)

    io = jax.lax.broadcasted_iota(jnp.int32, (1, _HALF), 1).astype(jnp.float32)
    freqs = jnp.exp(io * jnp.float32(-np.log(10000.0) / _HALF))
    args = ts_ref[...] * freqs
    t_emb = jnp.concatenate([jnp.cos(args), jnp.sin(args)], axis=-1)

    z1 = jnp.dot(t_emb, w1_ref[...], preferred_element_type=jnp.float32)
    z1 = z1 + b1_ref[...]
    z1 = z1 * jax.nn.sigmoid(z1)

    part = jnp.dot(z1, w2_ref[...], preferred_element_type=jnp.float32)

    @pl.when(k == 0)
    def _():
        out_ref[...] = part

    @pl.when(k > 0)
    def _():
        out_ref[...] = out_ref[...] + part


def _compute_emb(ts, time_W1, time_b1, time_W2):
    return pl.pallas_call(
        _emb_kernel,
        grid=(_NK,),
        in_specs=[
            pl.BlockSpec((_B, 1), lambda k: (0, 0)),
            pl.BlockSpec((_D, _KB), lambda k: (0, k)),
            pl.BlockSpec((1, _KB), lambda k: (0, k)),
            pl.BlockSpec((_KB, _T), lambda k: (k, 0)),
        ],
        out_specs=pl.BlockSpec((_B, _T), lambda k: (0, 0)),
        out_shape=jax.ShapeDtypeStruct((_B, _T), jnp.float32),
    )(ts, time_W1, time_b1.reshape(1, _T), time_W2)


def _tail_kernel(e_ref, b2_ref, embw_ref, embb_ref, wv_ref, wo_ref, bo_ref,
                 netw_ref, netb_ref, init_ref, tok_ref, pe_ref, col_ref):
    emb = e_ref[...] + b2_ref[...]
    s = emb * jax.nn.sigmoid(emb)
    h = jnp.dot(s, embw_ref[...], preferred_element_type=jnp.float32)
    h = h + embb_ref[...] + init_ref[...]
    v = jnp.dot(h, wv_ref[...], preferred_element_type=jnp.float32)
    x2 = jnp.dot(v, wo_ref[...], preferred_element_type=jnp.float32)
    x2 = x2 + bo_ref[...]
    pe = jnp.dot(x2, netw_ref[...], preferred_element_type=jnp.float32)
    pe_ref[...] = pe + netb_ref[...]
    io = jax.lax.broadcasted_iota(jnp.int32, (_B, _N), 1)
    col_ref[...] = jnp.max(
        jnp.where(tok_ref[...] == _PLACEHOLDER, io, 0), axis=1, keepdims=True)


def _compute_pe_cols(emb, time_b2, emb_W, emb_b, a2_Wv, a2_Wo, a2_bo,
                     net_W, net_b, init_emb, tok):
    full = lambda shape: pl.BlockSpec(shape, lambda: tuple(0 for _ in shape))
    return pl.pallas_call(
        _tail_kernel,
        in_specs=[
            full((_B, _T)), full((1, _T)), full((_T, _D)), full((1, _D)),
            full((_D, _INNER)), full((_INNER, _D)), full((1, _D)),
            full((_D, _D)), full((1, _D)), full((1, _D)), full((_B, _N)),
        ],
        out_specs=[full((_B, _D)), full((_B, 1))],
        out_shape=[jax.ShapeDtypeStruct((_B, _D), jnp.float32),
                   jax.ShapeDtypeStruct((_B, 1), jnp.int32)],
    )(emb, time_b2.reshape(1, _T), emb_W, emb_b.reshape(1, _D),
      a2_Wv, a2_Wo, a2_bo.reshape(1, _D), net_W, net_b.reshape(1, _D),
      init_emb, tok)


_NB = 7


def _assemble_kernel(tok_ref, pe_ref, emb_ref, out_ref):
    mask = tok_ref[...] == _PLACEHOLDER
    out_ref[...] = jnp.where(mask, pe_ref[...], emb_ref[...])


def _assemble_t(tok3, pe3, emb_t):
    return pl.pallas_call(
        _assemble_kernel,
        grid=(_N // _NB,),
        in_specs=[
            pl.BlockSpec((_NB, _B, 1), lambda i: (i, 0, 0)),
            pl.BlockSpec((1, _B, _D), lambda i: (0, 0, 0)),
            pl.BlockSpec((_NB, _B, _D), lambda i: (i, 0, 0)),
        ],
        out_specs=pl.BlockSpec((_NB, _B, _D), lambda i: (i, 0, 0)),
        out_shape=jax.ShapeDtypeStruct((_N, _B, _D), jnp.float32),
    )(tok3, pe3, emb_t)


_NW = 32
_G = _B * _N // 8
_CH = 80


def _sc_copy(src_flat):
    mesh = plsc.VectorSubcoreMesh(core_axis_name="c", subcore_axis_name="s")

    @functools.partial(
        pl.kernel,
        out_type=jax.ShapeDtypeStruct((_B * _N, _D), jnp.float32),
        mesh=mesh,
        scratch_types=[pltpu.VMEM((_CH, _D), jnp.float32),
                       pltpu.VMEM((_CH, _D), jnp.float32),
                       pltpu.SemaphoreType.DMA, pltpu.SemaphoreType.DMA,
                       pltpu.SemaphoreType.DMA, pltpu.SemaphoreType.DMA])
    def k(src_hbm, out_hbm, buf0, buf1, si0, si1, so0, so1):
        wid = jax.lax.axis_index("s") * 2 + jax.lax.axis_index("c")
        base = ((wid * _G) // _NW) * 8
        end = (((wid + 1) * _G) // _NW) * 8
        starts = [base, base + _CH, base + 2 * _CH, end - _CH]
        bufs, sis, sos = (buf0, buf1), (si0, si1), (so0, so1)
        ins = [pltpu.make_async_copy(src_hbm.at[pl.ds(starts[i], _CH)],
                                     bufs[i % 2], sis[i % 2])
               for i in range(4)]
        outs = [pltpu.make_async_copy(bufs[i % 2],
                                      out_hbm.at[pl.ds(starts[i], _CH)],
                                      sos[i % 2])
                for i in range(4)]
        ins[0].start()
        ins[1].start()
        ins[0].wait(); outs[0].start()
        ins[1].wait(); outs[1].start()
        outs[0].wait(); ins[2].start()
        outs[1].wait(); ins[3].start()
        ins[2].wait(); outs[2].start()
        ins[3].wait(); outs[3].start()
        outs[2].wait()
        outs[3].wait()

    return k(src_flat)


def _scatter_kernel(cols_ref, pe_ref, in_ref, out_ref, sem):
    del in_ref

    def body(b, carry):
        r = cols_ref[b] * _B + b
        pltpu.make_async_copy(pe_ref.at[b], out_ref.at[r], sem).start()
        return carry

    jax.lax.fori_loop(0, _B, body, 0)

    def drain(b, carry):
        pltpu.make_async_copy(pe_ref.at[0], out_ref.at[0], sem).wait()
        return carry

    jax.lax.fori_loop(0, _B, drain, 0)


def _scatter(cols, pe, out0):
    grid_spec = pltpu.PrefetchScalarGridSpec(
        num_scalar_prefetch=1,
        grid=(1,),
        in_specs=[pl.BlockSpec((_B, _D), lambda i, cols: (0, 0)),
                  pl.BlockSpec(memory_space=pl.ANY)],
        out_specs=pl.BlockSpec(memory_space=pl.ANY),
        scratch_shapes=[pltpu.SemaphoreType.DMA],
    )
    return pl.pallas_call(
        _scatter_kernel,
        grid_spec=grid_spec,
        out_shape=jax.ShapeDtypeStruct((_B * _N, _D), jnp.float32),
        input_output_aliases={2: 0},
    )(cols, pe, out0)


def kernel(tokenized_text, embedded_text, timestep, time_W1, time_b1,
           time_W2, time_b2, emb_W, emb_b, a1_Wq, a1_Wk, a1_Wv, a1_Wo, a1_bo,
           a2_Wq, a2_Wk, a2_Wv, a2_Wo, a2_bo, net_W, net_b, init_emb):
    ts = timestep.astype(jnp.float32).reshape(_B, 1)
    src_flat = embedded_text.transpose(1, 0, 2).reshape(_N * _B, _D)
    out0 = _sc_copy(src_flat)
    emb = _compute_emb(ts, time_W1, time_b1, time_W2)
    pe, cols = _compute_pe_cols(emb, time_b2, emb_W, emb_b, a2_Wv, a2_Wo,
                                a2_bo, net_W, net_b, init_emb, tokenized_text)
    out_flat = _scatter(cols.reshape(_B), pe, out0)
    return out_flat.reshape(_N, _B, _D).transpose(1, 0, 2)


# speedup vs baseline: 1.0320x; 1.0320x over previous
import functools

import numpy as np
import jax
import jax.numpy as jnp
from jax.experimental import pallas as pl
from jax.experimental.pallas import tpu as pltpu
from jax.experimental.pallas import tpu_sc as plsc

_PLACEHOLDER = 265
_B, _N, _D = 128, 77, 768
_T = 4 * _D
_INNER = 512
_KB = 512
_NK = _T // _KB
_HALF = _D // 2


def _pe_kernel(ts_ref, w1_ref, b1_ref, w2_ref, b2_ref, embw_ref, embb_ref,
               wv_ref, wo_ref, bo_ref, netw_ref, netb_ref, init_ref, tok_ref,
               pe_ref, col_ref, acc_ref):
    k = pl.program_id(0)

    io = jax.lax.broadcasted_iota(jnp.int32, (1, _HALF), 1).astype(jnp.float32)
    freqs = jnp.exp(io * jnp.float32(-np.log(10000.0) / _HALF))
    args = ts_ref[...] * freqs
    t_emb = jnp.concatenate([jnp.cos(args), jnp.sin(args)], axis=-1)

    z1 = jnp.dot(t_emb, w1_ref[...], preferred_element_type=jnp.float32)
    z1 = z1 + b1_ref[...]
    z1 = z1 * jax.nn.sigmoid(z1)

    part = jnp.dot(z1, w2_ref[...], preferred_element_type=jnp.float32)

    @pl.when(k == 0)
    def _():
        acc_ref[...] = part

    @pl.when(k > 0)
    def _():
        acc_ref[...] = acc_ref[...] + part

    @pl.when(k == _NK - 1)
    def _():
        emb = acc_ref[...] + b2_ref[...]
        s = emb * jax.nn.sigmoid(emb)
        h = jnp.dot(s, embw_ref[...], preferred_element_type=jnp.float32)
        h = h + embb_ref[...] + init_ref[...]
        v = jnp.dot(h, wv_ref[...], preferred_element_type=jnp.float32)
        x2 = jnp.dot(v, wo_ref[...], preferred_element_type=jnp.float32)
        x2 = x2 + bo_ref[...]
        pe = jnp.dot(x2, netw_ref[...], preferred_element_type=jnp.float32)
        pe_ref[...] = pe + netb_ref[...]
        io2 = jax.lax.broadcasted_iota(jnp.int32, (_B, _N), 1)
        col_ref[...] = jnp.max(
            jnp.where(tok_ref[...] == _PLACEHOLDER, io2, 0),
            axis=1, keepdims=True)


def _compute_pe_cols(ts, time_W1, time_b1, time_W2, time_b2, emb_W, emb_b,
                     a2_Wv, a2_Wo, a2_bo, net_W, net_b, init_emb, tok):
    const = lambda shape: pl.BlockSpec(shape, lambda k: tuple(0 for _ in shape))
    return pl.pallas_call(
        _pe_kernel,
        grid=(_NK,),
        in_specs=[
            const((_B, 1)),
            pl.BlockSpec((_D, _KB), lambda k: (0, k)),
            pl.BlockSpec((1, _KB), lambda k: (0, k)),
            pl.BlockSpec((_KB, _T), lambda k: (k, 0)),
            const((1, _T)), const((_T, _D)), const((1, _D)),
            const((_D, _INNER)), const((_INNER, _D)), const((1, _D)),
            const((_D, _D)), const((1, _D)), const((1, _D)), const((_B, _N)),
        ],
        out_specs=[const((_B, _D)), const((_B, 1))],
        out_shape=[jax.ShapeDtypeStruct((_B, _D), jnp.float32),
                   jax.ShapeDtypeStruct((_B, 1), jnp.int32)],
        scratch_shapes=[pltpu.VMEM((_B, _T), jnp.float32)],
    )(ts, time_W1, time_b1.reshape(1, _T), time_W2, time_b2.reshape(1, _T),
      emb_W, emb_b.reshape(1, _D), a2_Wv, a2_Wo, a2_bo.reshape(1, _D),
      net_W, net_b.reshape(1, _D), init_emb, tok)


_NB = 7


def _assemble_kernel(tok_ref, pe_ref, emb_ref, out_ref):
    mask = tok_ref[...] == _PLACEHOLDER
    out_ref[...] = jnp.where(mask, pe_ref[...], emb_ref[...])


def _assemble_t(tok3, pe3, emb_t):
    return pl.pallas_call(
        _assemble_kernel,
        grid=(_N // _NB,),
        in_specs=[
            pl.BlockSpec((_NB, _B, 1), lambda i: (i, 0, 0)),
            pl.BlockSpec((1, _B, _D), lambda i: (0, 0, 0)),
            pl.BlockSpec((_NB, _B, _D), lambda i: (i, 0, 0)),
        ],
        out_specs=pl.BlockSpec((_NB, _B, _D), lambda i: (i, 0, 0)),
        out_shape=jax.ShapeDtypeStruct((_N, _B, _D), jnp.float32),
    )(tok3, pe3, emb_t)


_NW = 32
_G = _B * _N // 8
_CH = 80


def _sc_copy(src_flat):
    mesh = plsc.VectorSubcoreMesh(core_axis_name="c", subcore_axis_name="s")

    @functools.partial(
        pl.kernel,
        out_type=jax.ShapeDtypeStruct((_B * _N, _D), jnp.float32),
        mesh=mesh,
        scratch_types=[pltpu.VMEM((_CH, _D), jnp.float32),
                       pltpu.VMEM((_CH, _D), jnp.float32),
                       pltpu.SemaphoreType.DMA, pltpu.SemaphoreType.DMA,
                       pltpu.SemaphoreType.DMA, pltpu.SemaphoreType.DMA])
    def k(src_hbm, out_hbm, buf0, buf1, si0, si1, so0, so1):
        wid = jax.lax.axis_index("s") * 2 + jax.lax.axis_index("c")
        base = ((wid * _G) // _NW) * 8
        end = (((wid + 1) * _G) // _NW) * 8
        starts = [base, base + _CH, base + 2 * _CH, end - _CH]
        bufs, sis, sos = (buf0, buf1), (si0, si1), (so0, so1)
        ins = [pltpu.make_async_copy(src_hbm.at[pl.ds(starts[i], _CH)],
                                     bufs[i % 2], sis[i % 2])
               for i in range(4)]
        outs = [pltpu.make_async_copy(bufs[i % 2],
                                      out_hbm.at[pl.ds(starts[i], _CH)],
                                      sos[i % 2])
                for i in range(4)]
        ins[0].start()
        ins[1].start()
        ins[0].wait(); outs[0].start()
        ins[1].wait(); outs[1].start()
        outs[0].wait(); ins[2].start()
        outs[1].wait(); ins[3].start()
        ins[2].wait(); outs[2].start()
        ins[3].wait(); outs[3].start()
        outs[2].wait()
        outs[3].wait()

    return k(src_flat)


def _scatter_kernel(cols_ref, pe_ref, in_ref, out_ref, sem):
    del in_ref

    def body(b, carry):
        r = cols_ref[b, 0] * _B + b
        pltpu.make_async_copy(pe_ref.at[b], out_ref.at[r], sem).start()
        return carry

    jax.lax.fori_loop(0, _B, body, 0)

    def drain(b, carry):
        pltpu.make_async_copy(pe_ref.at[0], out_ref.at[0], sem).wait()
        return carry

    jax.lax.fori_loop(0, _B, drain, 0)


def _scatter(cols, pe, out0):
    grid_spec = pltpu.PrefetchScalarGridSpec(
        num_scalar_prefetch=1,
        grid=(1,),
        in_specs=[pl.BlockSpec((_B, _D), lambda i, cols_sp: (0, 0)),
                  pl.BlockSpec(memory_space=pl.ANY)],
        out_specs=pl.BlockSpec(memory_space=pl.ANY),
        scratch_shapes=[pltpu.SemaphoreType.DMA],
    )
    return pl.pallas_call(
        _scatter_kernel,
        grid_spec=grid_spec,
        out_shape=jax.ShapeDtypeStruct((_B * _N, _D), jnp.float32),
        input_output_aliases={2: 0},
    )(cols, pe, out0)


def kernel(tokenized_text, embedded_text, timestep, time_W1, time_b1,
           time_W2, time_b2, emb_W, emb_b, a1_Wq, a1_Wk, a1_Wv, a1_Wo, a1_bo,
           a2_Wq, a2_Wk, a2_Wv, a2_Wo, a2_bo, net_W, net_b, init_emb):
    ts = timestep.astype(jnp.float32).reshape(_B, 1)
    src_flat = embedded_text.transpose(1, 0, 2).reshape(_N * _B, _D)
    out0 = _sc_copy(src_flat)
    pe, cols = _compute_pe_cols(ts, time_W1, time_b1, time_W2, time_b2,
                                emb_W, emb_b, a2_Wv, a2_Wo, a2_bo,
                                net_W, net_b, init_emb, tokenized_text)
    out_flat = _scatter(cols, pe, out0)
    return out_flat.reshape(_N, _B, _D).transpose(1, 0, 2)


# speedup vs baseline: 1.1673x; 1.1311x over previous
import functools

import numpy as np
import jax
import jax.numpy as jnp
from jax.experimental import pallas as pl
from jax.experimental.pallas import tpu as pltpu
from jax.experimental.pallas import tpu_sc as plsc

_PLACEHOLDER = 265
_B, _N, _D = 128, 77, 768
_T = 4 * _D
_INNER = 512
_KB = 512
_NK = _T // _KB
_HALF = _D // 2


def _pe_kernel(ts_ref, w1_ref, b1_ref, w2_ref, b2_ref, embw_ref, embb_ref,
               wv_ref, wo_ref, bo_ref, netw_ref, netb_ref, init_ref, tok_ref,
               pe_ref, col_ref, acc_ref):
    k = pl.program_id(0)

    io = jax.lax.broadcasted_iota(jnp.int32, (1, _HALF), 1).astype(jnp.float32)
    freqs = jnp.exp(io * jnp.float32(-np.log(10000.0) / _HALF))
    args = ts_ref[...] * freqs
    t_emb = jnp.concatenate([jnp.cos(args), jnp.sin(args)], axis=-1)

    z1 = jnp.dot(t_emb, w1_ref[...], preferred_element_type=jnp.float32)
    z1 = z1 + b1_ref[...]
    z1 = z1 * jax.nn.sigmoid(z1)

    part = jnp.dot(z1, w2_ref[...], preferred_element_type=jnp.float32)

    @pl.when(k == 0)
    def _():
        acc_ref[...] = part

    @pl.when(k > 0)
    def _():
        acc_ref[...] = acc_ref[...] + part

    @pl.when(k == _NK - 1)
    def _():
        emb = acc_ref[...] + b2_ref[...]
        s = emb * jax.nn.sigmoid(emb)
        h = jnp.dot(s, embw_ref[...], preferred_element_type=jnp.float32)
        h = h + embb_ref[...] + init_ref[...]
        v = jnp.dot(h, wv_ref[...], preferred_element_type=jnp.float32)
        x2 = jnp.dot(v, wo_ref[...], preferred_element_type=jnp.float32)
        x2 = x2 + bo_ref[...]
        pe = jnp.dot(x2, netw_ref[...], preferred_element_type=jnp.float32)
        pe_ref[...] = pe + netb_ref[...]
        io2 = jax.lax.broadcasted_iota(jnp.int32, (_B, _N), 1)
        col_ref[...] = jnp.max(
            jnp.where(tok_ref[...] == _PLACEHOLDER, io2, 0),
            axis=1, keepdims=True)


def _compute_pe_cols(ts, time_W1, time_b1, time_W2, time_b2, emb_W, emb_b,
                     a2_Wv, a2_Wo, a2_bo, net_W, net_b, init_emb, tok):
    const = lambda shape: pl.BlockSpec(shape, lambda k: tuple(0 for _ in shape))
    return pl.pallas_call(
        _pe_kernel,
        grid=(_NK,),
        in_specs=[
            const((_B, 1)),
            pl.BlockSpec((_D, _KB), lambda k: (0, k)),
            pl.BlockSpec((1, _KB), lambda k: (0, k)),
            pl.BlockSpec((_KB, _T), lambda k: (k, 0)),
            const((1, _T)), const((_T, _D)), const((1, _D)),
            const((_D, _INNER)), const((_INNER, _D)), const((1, _D)),
            const((_D, _D)), const((1, _D)), const((1, _D)), const((_B, _N)),
        ],
        out_specs=[const((_B, _D)), const((_B, 1))],
        out_shape=[jax.ShapeDtypeStruct((_B, _D), jnp.float32),
                   jax.ShapeDtypeStruct((_B, 1), jnp.int32)],
        scratch_shapes=[pltpu.VMEM((_B, _T), jnp.float32)],
    )(ts, time_W1, time_b1.reshape(1, _T), time_W2, time_b2.reshape(1, _T),
      emb_W, emb_b.reshape(1, _D), a2_Wv, a2_Wo, a2_bo.reshape(1, _D),
      net_W, net_b.reshape(1, _D), init_emb, tok)


_NB = 7


def _assemble_kernel(tok_ref, pe_ref, emb_ref, out_ref):
    mask = tok_ref[...] == _PLACEHOLDER
    out_ref[...] = jnp.where(mask, pe_ref[...], emb_ref[...])


def _assemble_t(tok3, pe3, emb_t):
    return pl.pallas_call(
        _assemble_kernel,
        grid=(_N // _NB,),
        in_specs=[
            pl.BlockSpec((_NB, _B, 1), lambda i: (i, 0, 0)),
            pl.BlockSpec((1, _B, _D), lambda i: (0, 0, 0)),
            pl.BlockSpec((_NB, _B, _D), lambda i: (i, 0, 0)),
        ],
        out_specs=pl.BlockSpec((_NB, _B, _D), lambda i: (i, 0, 0)),
        out_shape=jax.ShapeDtypeStruct((_N, _B, _D), jnp.float32),
    )(tok3, pe3, emb_t)


_NW = 32
_G = _B * _N // 8
_CH = 80


def _sc_copy(src_flat):
    mesh = plsc.VectorSubcoreMesh(core_axis_name="c", subcore_axis_name="s")

    @functools.partial(
        pl.kernel,
        out_type=jax.ShapeDtypeStruct((_B * _N, _D), jnp.float32),
        mesh=mesh,
        scratch_types=[pltpu.VMEM((_CH, _D), jnp.float32),
                       pltpu.VMEM((_CH, _D), jnp.float32),
                       pltpu.SemaphoreType.DMA, pltpu.SemaphoreType.DMA,
                       pltpu.SemaphoreType.DMA, pltpu.SemaphoreType.DMA])
    def k(src_hbm, out_hbm, buf0, buf1, si0, si1, so0, so1):
        wid = jax.lax.axis_index("s") * 2 + jax.lax.axis_index("c")
        base = ((wid * _G) // _NW) * 8
        end = (((wid + 1) * _G) // _NW) * 8
        starts = [base, base + _CH, base + 2 * _CH, end - _CH]
        bufs, sis, sos = (buf0, buf1), (si0, si1), (so0, so1)
        ins = [pltpu.make_async_copy(src_hbm.at[pl.ds(starts[i], _CH)],
                                     bufs[i % 2], sis[i % 2])
               for i in range(4)]
        outs = [pltpu.make_async_copy(bufs[i % 2],
                                      out_hbm.at[pl.ds(starts[i], _CH)],
                                      sos[i % 2])
                for i in range(4)]
        ins[0].start()
        ins[1].start()
        ins[0].wait(); outs[0].start()
        ins[1].wait(); outs[1].start()
        outs[0].wait(); ins[2].start()
        outs[1].wait(); ins[3].start()
        ins[2].wait(); outs[2].start()
        ins[3].wait(); outs[3].start()
        outs[2].wait()
        outs[3].wait()

    return k(src_flat)


def _scatter_kernel(cols_ref, pe_ref, in_ref, out_ref, sem):
    del in_ref

    def body(b, carry):
        r = cols_ref[b, 0] * _B + b
        pltpu.make_async_copy(pe_ref.at[b], out_ref.at[r], sem).start()
        return carry

    jax.lax.fori_loop(0, _B, body, 0)

    def drain(b, carry):
        pltpu.make_async_copy(pe_ref.at[0], out_ref.at[0], sem).wait()
        return carry

    jax.lax.fori_loop(0, _B, drain, 0)


def _scatter(cols, pe, out0):
    grid_spec = pltpu.PrefetchScalarGridSpec(
        num_scalar_prefetch=1,
        grid=(1,),
        in_specs=[pl.BlockSpec((_B, _D), lambda i, cols_sp: (0, 0)),
                  pl.BlockSpec(memory_space=pl.ANY)],
        out_specs=pl.BlockSpec(memory_space=pl.ANY),
        scratch_shapes=[pltpu.SemaphoreType.DMA],
    )
    return pl.pallas_call(
        _scatter_kernel,
        grid_spec=grid_spec,
        out_shape=jax.ShapeDtypeStruct((_B * _N, _D), jnp.float32),
        input_output_aliases={2: 0},
    )(cols, pe, out0)


def kernel(tokenized_text, embedded_text, timestep, time_W1, time_b1,
           time_W2, time_b2, emb_W, emb_b, a1_Wq, a1_Wk, a1_Wv, a1_Wo, a1_bo,
           a2_Wq, a2_Wk, a2_Wv, a2_Wo, a2_bo, net_W, net_b, init_emb):
    ts = timestep.astype(jnp.float32).reshape(_B, 1)
    pe, _cols = _compute_pe_cols(ts, time_W1, time_b1, time_W2, time_b2,
                                 emb_W, emb_b, a2_Wv, a2_Wo, a2_bo,
                                 net_W, net_b, init_emb, tokenized_text)
    out_t = _assemble_t(tokenized_text.T.reshape(_N, _B, 1),
                        pe.reshape(1, _B, _D),
                        embedded_text.transpose(1, 0, 2))
    return out_t.transpose(1, 0, 2)


# speedup vs baseline: 1.4167x; 1.2136x over previous
import functools

import numpy as np
import jax
import jax.numpy as jnp
from jax.experimental import pallas as pl
from jax.experimental.pallas import tpu as pltpu
from jax.experimental.pallas import tpu_sc as plsc

_PLACEHOLDER = 265
_B, _N, _D = 128, 77, 768
_T = 4 * _D
_INNER = 512
_KB = 512
_NK = _T // _KB
_HALF = _D // 2


def _pe_kernel(ts_ref, w1_ref, b1_ref, w2_ref, b2_ref, embw_ref, embb_ref,
               wv_ref, wo_ref, bo_ref, netw_ref, netb_ref, init_ref, tok_ref,
               pe_ref, col_ref, acc_ref):
    k = pl.program_id(0)

    io = jax.lax.broadcasted_iota(jnp.int32, (1, _HALF), 1).astype(jnp.float32)
    freqs = jnp.exp(io * jnp.float32(-np.log(10000.0) / _HALF))
    args = ts_ref[...] * freqs
    t_emb = jnp.concatenate([jnp.cos(args), jnp.sin(args)], axis=-1)

    z1 = jnp.dot(t_emb, w1_ref[...], preferred_element_type=jnp.float32)
    z1 = z1 + b1_ref[...]
    z1 = z1 * jax.nn.sigmoid(z1)

    part = jnp.dot(z1, w2_ref[...], preferred_element_type=jnp.float32)

    @pl.when(k == 0)
    def _():
        acc_ref[...] = part

    @pl.when(k > 0)
    def _():
        acc_ref[...] = acc_ref[...] + part

    @pl.when(k == _NK - 1)
    def _():
        emb = acc_ref[...] + b2_ref[...]
        s = emb * jax.nn.sigmoid(emb)
        h = jnp.dot(s, embw_ref[...], preferred_element_type=jnp.float32)
        h = h + embb_ref[...] + init_ref[...]
        v = jnp.dot(h, wv_ref[...], preferred_element_type=jnp.float32)
        x2 = jnp.dot(v, wo_ref[...], preferred_element_type=jnp.float32)
        x2 = x2 + bo_ref[...]
        pe = jnp.dot(x2, netw_ref[...], preferred_element_type=jnp.float32)
        pe_ref[...] = pe + netb_ref[...]
        io2 = jax.lax.broadcasted_iota(jnp.int32, (_B, _N), 1)
        col_ref[...] = jnp.max(
            jnp.where(tok_ref[...] == _PLACEHOLDER, io2, 0),
            axis=1, keepdims=True)


def _compute_pe_cols(ts, time_W1, time_b1, time_W2, time_b2, emb_W, emb_b,
                     a2_Wv, a2_Wo, a2_bo, net_W, net_b, init_emb, tok):
    const = lambda shape: pl.BlockSpec(shape, lambda k: tuple(0 for _ in shape))
    return pl.pallas_call(
        _pe_kernel,
        grid=(_NK,),
        in_specs=[
            const((_B, 1)),
            pl.BlockSpec((_D, _KB), lambda k: (0, k)),
            pl.BlockSpec((_KB,), lambda k: (k,)),
            pl.BlockSpec((_KB, _T), lambda k: (k, 0)),
            const((_T,)), const((_T, _D)), const((_D,)),
            const((_D, _INNER)), const((_INNER, _D)), const((_D,)),
            const((_D, _D)), const((_D,)), const((1, _D)), const((_B, _N)),
        ],
        out_specs=[const((_B, _D)), const((_B, 1))],
        out_shape=[jax.ShapeDtypeStruct((_B, _D), jnp.float32),
                   jax.ShapeDtypeStruct((_B, 1), jnp.int32)],
        scratch_shapes=[pltpu.VMEM((_B, _T), jnp.float32)],
    )(ts, time_W1, time_b1, time_W2, time_b2,
      emb_W, emb_b, a2_Wv, a2_Wo, a2_bo,
      net_W, net_b, init_emb, tok)


_NB = 11


def _assemble_kernel(cols_ref, pe_ref, emb_ref, out_ref):
    i = pl.program_id(0)
    n = jax.lax.broadcasted_iota(jnp.int32, (_NB, _B, 1), 0) + i * _NB
    mask = cols_ref[...] == n
    out_ref[...] = jnp.where(mask, pe_ref[...], emb_ref[...])


def _assemble_t(cols3, pe3, emb_t):
    return pl.pallas_call(
        _assemble_kernel,
        grid=(_N // _NB,),
        in_specs=[
            pl.BlockSpec((1, _B, 1), lambda i: (0, 0, 0)),
            pl.BlockSpec((1, _B, _D), lambda i: (0, 0, 0)),
            pl.BlockSpec((_NB, _B, _D), lambda i: (i, 0, 0)),
        ],
        out_specs=pl.BlockSpec((_NB, _B, _D), lambda i: (i, 0, 0)),
        out_shape=jax.ShapeDtypeStruct((_N, _B, _D), jnp.float32),
    )(cols3, pe3, emb_t)


_NW = 32
_G = _B * _N // 8
_CH = 80


def _sc_copy(src_flat):
    mesh = plsc.VectorSubcoreMesh(core_axis_name="c", subcore_axis_name="s")

    @functools.partial(
        pl.kernel,
        out_type=jax.ShapeDtypeStruct((_B * _N, _D), jnp.float32),
        mesh=mesh,
        scratch_types=[pltpu.VMEM((_CH, _D), jnp.float32),
                       pltpu.VMEM((_CH, _D), jnp.float32),
                       pltpu.SemaphoreType.DMA, pltpu.SemaphoreType.DMA,
                       pltpu.SemaphoreType.DMA, pltpu.SemaphoreType.DMA])
    def k(src_hbm, out_hbm, buf0, buf1, si0, si1, so0, so1):
        wid = jax.lax.axis_index("s") * 2 + jax.lax.axis_index("c")
        base = ((wid * _G) // _NW) * 8
        end = (((wid + 1) * _G) // _NW) * 8
        starts = [base, base + _CH, base + 2 * _CH, end - _CH]
        bufs, sis, sos = (buf0, buf1), (si0, si1), (so0, so1)
        ins = [pltpu.make_async_copy(src_hbm.at[pl.ds(starts[i], _CH)],
                                     bufs[i % 2], sis[i % 2])
               for i in range(4)]
        outs = [pltpu.make_async_copy(bufs[i % 2],
                                      out_hbm.at[pl.ds(starts[i], _CH)],
                                      sos[i % 2])
                for i in range(4)]
        ins[0].start()
        ins[1].start()
        ins[0].wait(); outs[0].start()
        ins[1].wait(); outs[1].start()
        outs[0].wait(); ins[2].start()
        outs[1].wait(); ins[3].start()
        ins[2].wait(); outs[2].start()
        ins[3].wait(); outs[3].start()
        outs[2].wait()
        outs[3].wait()

    return k(src_flat)


def _scatter_kernel(cols_ref, pe_ref, in_ref, out_ref, sem):
    del in_ref

    def body(b, carry):
        r = cols_ref[b, 0] * _B + b
        pltpu.make_async_copy(pe_ref.at[b], out_ref.at[r], sem).start()
        return carry

    jax.lax.fori_loop(0, _B, body, 0)

    def drain(b, carry):
        pltpu.make_async_copy(pe_ref.at[0], out_ref.at[0], sem).wait()
        return carry

    jax.lax.fori_loop(0, _B, drain, 0)


def _scatter(cols, pe, out0):
    grid_spec = pltpu.PrefetchScalarGridSpec(
        num_scalar_prefetch=1,
        grid=(1,),
        in_specs=[pl.BlockSpec((_B, _D), lambda i, cols_sp: (0, 0)),
                  pl.BlockSpec(memory_space=pl.ANY)],
        out_specs=pl.BlockSpec(memory_space=pl.ANY),
        scratch_shapes=[pltpu.SemaphoreType.DMA],
    )
    return pl.pallas_call(
        _scatter_kernel,
        grid_spec=grid_spec,
        out_shape=jax.ShapeDtypeStruct((_B * _N, _D), jnp.float32),
        input_output_aliases={2: 0},
    )(cols, pe, out0)


def kernel(tokenized_text, embedded_text, timestep, time_W1, time_b1,
           time_W2, time_b2, emb_W, emb_b, a1_Wq, a1_Wk, a1_Wv, a1_Wo, a1_bo,
           a2_Wq, a2_Wk, a2_Wv, a2_Wo, a2_bo, net_W, net_b, init_emb):
    ts = timestep.astype(jnp.float32).reshape(_B, 1)
    pe, cols = _compute_pe_cols(ts, time_W1, time_b1, time_W2, time_b2,
                                emb_W, emb_b, a2_Wv, a2_Wo, a2_bo,
                                net_W, net_b, init_emb, tokenized_text)
    out_t = _assemble_t(cols.reshape(1, _B, 1),
                        pe.reshape(1, _B, _D),
                        embedded_text.transpose(1, 0, 2))
    return out_t.transpose(1, 0, 2)


# speedup vs baseline: 1.4283x; 1.0082x over previous
import functools

import numpy as np
import jax
import jax.numpy as jnp
from jax.experimental import pallas as pl
from jax.experimental.pallas import tpu as pltpu
from jax.experimental.pallas import tpu_sc as plsc

_PLACEHOLDER = 265
_B, _N, _D = 128, 77, 768
_T = 4 * _D
_INNER = 512
_KB = 512
_NK = _T // _KB
_HALF = _D // 2


def _pe_kernel(ts_ref, w1_ref, b1_ref, w2_ref, b2_ref, embw_ref, embb_ref,
               wv_ref, wo_ref, bo_ref, netw_ref, netb_ref, init_ref, tok_ref,
               pe_ref, col_ref, acc_ref):
    k = pl.program_id(0)

    io = jax.lax.broadcasted_iota(jnp.int32, (1, _HALF), 1).astype(jnp.float32)
    freqs = jnp.exp(io * jnp.float32(-np.log(10000.0) / _HALF))
    args = ts_ref[...] * freqs
    t_emb = jnp.concatenate([jnp.cos(args), jnp.sin(args)], axis=-1)

    z1 = jnp.dot(t_emb, w1_ref[...], preferred_element_type=jnp.float32)
    z1 = z1 + b1_ref[...]
    z1 = z1 * jax.nn.sigmoid(z1)

    part = jnp.dot(z1, w2_ref[...], preferred_element_type=jnp.float32)

    @pl.when(k == 0)
    def _():
        acc_ref[...] = part

    @pl.when(k > 0)
    def _():
        acc_ref[...] = acc_ref[...] + part

    @pl.when(k == _NK - 1)
    def _():
        emb = acc_ref[...] + b2_ref[...]
        s = emb * jax.nn.sigmoid(emb)
        h = jnp.dot(s, embw_ref[...], preferred_element_type=jnp.float32)
        h = h + embb_ref[...] + init_ref[...]
        v = jnp.dot(h, wv_ref[...], preferred_element_type=jnp.float32)
        x2 = jnp.dot(v, wo_ref[...], preferred_element_type=jnp.float32)
        x2 = x2 + bo_ref[...]
        pe = jnp.dot(x2, netw_ref[...], preferred_element_type=jnp.float32)
        pe_ref[...] = pe + netb_ref[...]
        io2 = jax.lax.broadcasted_iota(jnp.int32, (_N, _B), 0)
        col_ref[...] = jnp.max(
            jnp.where(tok_ref[...] == _PLACEHOLDER, io2, 0),
            axis=0, keepdims=True)


def _compute_pe_cols(ts, time_W1, time_b1, time_W2, time_b2, emb_W, emb_b,
                     a2_Wv, a2_Wo, a2_bo, net_W, net_b, init_emb, tok):
    const = lambda shape: pl.BlockSpec(shape, lambda k: tuple(0 for _ in shape))
    return pl.pallas_call(
        _pe_kernel,
        grid=(_NK,),
        in_specs=[
            const((_B, 1)),
            pl.BlockSpec((_D, _KB), lambda k: (0, k)),
            pl.BlockSpec((_KB,), lambda k: (k,)),
            pl.BlockSpec((_KB, _T), lambda k: (k, 0)),
            const((_T,)), const((_T, _D)), const((_D,)),
            const((_D, _INNER)), const((_INNER, _D)), const((_D,)),
            const((_D, _D)), const((_D,)), const((1, _D)), const((_N, _B)),
        ],
        out_specs=[const((_B, _D)), const((1, _B))],
        out_shape=[jax.ShapeDtypeStruct((_B, _D), jnp.float32),
                   jax.ShapeDtypeStruct((1, _B), jnp.int32)],
        scratch_shapes=[pltpu.VMEM((_B, _T), jnp.float32)],
    )(ts, time_W1, time_b1, time_W2, time_b2,
      emb_W, emb_b, a2_Wv, a2_Wo, a2_bo,
      net_W, net_b, init_emb, tok)


_NB = 11


def _assemble_kernel(cols_ref, pe_ref, emb_ref, out_ref):
    i = pl.program_id(0)
    n = jax.lax.broadcasted_iota(jnp.int32, (_NB, _B, 1), 0) + i * _NB
    mask = cols_ref[...] == n
    out_ref[...] = jnp.where(mask, pe_ref[...], emb_ref[...])


def _assemble_t(cols3, pe3, emb_t):
    return pl.pallas_call(
        _assemble_kernel,
        grid=(_N // _NB,),
        in_specs=[
            pl.BlockSpec((1, _B, 1), lambda i: (0, 0, 0)),
            pl.BlockSpec((1, _B, _D), lambda i: (0, 0, 0)),
            pl.BlockSpec((_NB, _B, _D), lambda i: (i, 0, 0)),
        ],
        out_specs=pl.BlockSpec((_NB, _B, _D), lambda i: (i, 0, 0)),
        out_shape=jax.ShapeDtypeStruct((_N, _B, _D), jnp.float32),
    )(cols3, pe3, emb_t)


_NW = 32
_G = _B * _N // 8
_CH = 80


def _sc_copy(src_flat):
    mesh = plsc.VectorSubcoreMesh(core_axis_name="c", subcore_axis_name="s")

    @functools.partial(
        pl.kernel,
        out_type=jax.ShapeDtypeStruct((_B * _N, _D), jnp.float32),
        mesh=mesh,
        scratch_types=[pltpu.VMEM((_CH, _D), jnp.float32),
                       pltpu.VMEM((_CH, _D), jnp.float32),
                       pltpu.SemaphoreType.DMA, pltpu.SemaphoreType.DMA,
                       pltpu.SemaphoreType.DMA, pltpu.SemaphoreType.DMA])
    def k(src_hbm, out_hbm, buf0, buf1, si0, si1, so0, so1):
        wid = jax.lax.axis_index("s") * 2 + jax.lax.axis_index("c")
        base = ((wid * _G) // _NW) * 8
        end = (((wid + 1) * _G) // _NW) * 8
        starts = [base, base + _CH, base + 2 * _CH, end - _CH]
        bufs, sis, sos = (buf0, buf1), (si0, si1), (so0, so1)
        ins = [pltpu.make_async_copy(src_hbm.at[pl.ds(starts[i], _CH)],
                                     bufs[i % 2], sis[i % 2])
               for i in range(4)]
        outs = [pltpu.make_async_copy(bufs[i % 2],
                                      out_hbm.at[pl.ds(starts[i], _CH)],
                                      sos[i % 2])
                for i in range(4)]
        ins[0].start()
        ins[1].start()
        ins[0].wait(); outs[0].start()
        ins[1].wait(); outs[1].start()
        outs[0].wait(); ins[2].start()
        outs[1].wait(); ins[3].start()
        ins[2].wait(); outs[2].start()
        ins[3].wait(); outs[3].start()
        outs[2].wait()
        outs[3].wait()

    return k(src_flat)


def _scatter_kernel(cols_ref, pe_ref, in_ref, out_ref, sem):
    del in_ref

    def body(b, carry):
        r = cols_ref[b, 0] * _B + b
        pltpu.make_async_copy(pe_ref.at[b], out_ref.at[r], sem).start()
        return carry

    jax.lax.fori_loop(0, _B, body, 0)

    def drain(b, carry):
        pltpu.make_async_copy(pe_ref.at[0], out_ref.at[0], sem).wait()
        return carry

    jax.lax.fori_loop(0, _B, drain, 0)


def _scatter(cols, pe, out0):
    grid_spec = pltpu.PrefetchScalarGridSpec(
        num_scalar_prefetch=1,
        grid=(1,),
        in_specs=[pl.BlockSpec((_B, _D), lambda i, cols_sp: (0, 0)),
                  pl.BlockSpec(memory_space=pl.ANY)],
        out_specs=pl.BlockSpec(memory_space=pl.ANY),
        scratch_shapes=[pltpu.SemaphoreType.DMA],
    )
    return pl.pallas_call(
        _scatter_kernel,
        grid_spec=grid_spec,
        out_shape=jax.ShapeDtypeStruct((_B * _N, _D), jnp.float32),
        input_output_aliases={2: 0},
    )(cols, pe, out0)


def kernel(tokenized_text, embedded_text, timestep, time_W1, time_b1,
           time_W2, time_b2, emb_W, emb_b, a1_Wq, a1_Wk, a1_Wv, a1_Wo, a1_bo,
           a2_Wq, a2_Wk, a2_Wv, a2_Wo, a2_bo, net_W, net_b, init_emb):
    ts = timestep.astype(jnp.float32).reshape(_B, 1)
    pe, cols = _compute_pe_cols(ts, time_W1, time_b1, time_W2, time_b2,
                                emb_W, emb_b, a2_Wv, a2_Wo, a2_bo,
                                net_W, net_b, init_emb, tokenized_text.T)
    out_t = _assemble_t(cols.reshape(1, _B, 1),
                        pe.reshape(1, _B, _D),
                        embedded_text.transpose(1, 0, 2))
    return out_t.transpose(1, 0, 2)
